# baseline (speedup 1.0000x reference)
import jax
import jax.numpy as jnp
from jax.experimental import pallas as pl
from jax.experimental.pallas import tpu as pltpu

_ROWS = 10000
_FEAT = 256
_BLOCK_ROWS = 400


def _gelu_exact(x):
    return 0.5 * x * (1.0 + jax.lax.erf(x * 0.7071067811865476))


def _double_gelu_kernel(x_ref, o_ref):
    o_ref[...] = _gelu_exact(_gelu_exact(x_ref[...]))


def kernel(embs, edge_index, batch_size):
    del edge_index, batch_size
    n, d = embs.shape
    block_rows = _BLOCK_ROWS if (n == _ROWS and d == _FEAT) else n
    grid = (n // block_rows,) if n % block_rows == 0 else (1,)
    if n % block_rows != 0:
        block_rows = n
    return pl.pallas_call(
        _double_gelu_kernel,
        grid=grid,
        in_specs=[pl.BlockSpec((block_rows, d), lambda i: (i, 0))],
        out_specs=pl.BlockSpec((block_rows, d), lambda i: (i, 0)),
        out_shape=jax.ShapeDtypeStruct((n, d), embs.dtype),
        compiler_params=pltpu.CompilerParams(
            dimension_semantics=("parallel",),
        ),
    )(embs)


# speedup vs baseline: 1.8244x; 1.8244x over previous
import jax
import jax.numpy as jnp
from jax.experimental import pallas as pl
from jax.experimental.pallas import tpu as pltpu

_ROWS = 10000
_FEAT = 256
_BLOCK_ROWS = 2000


def _gelu_exact(x):
    return 0.5 * x * (1.0 + jax.lax.erf(x * 0.7071067811865476))


def _double_gelu_kernel(x_ref, o_ref):
    o_ref[...] = _gelu_exact(_gelu_exact(x_ref[...]))


def kernel(embs, edge_index, batch_size):
    del edge_index, batch_size
    n, d = embs.shape
    block_rows = _BLOCK_ROWS if (n == _ROWS and d == _FEAT) else n
    grid = (n // block_rows,) if n % block_rows == 0 else (1,)
    if n % block_rows != 0:
        block_rows = n
    return pl.pallas_call(
        _double_gelu_kernel,
        grid=grid,
        in_specs=[pl.BlockSpec((block_rows, d), lambda i: (i, 0))],
        out_specs=pl.BlockSpec((block_rows, d), lambda i: (i, 0)),
        out_shape=jax.ShapeDtypeStruct((n, d), embs.dtype),
        compiler_params=pltpu.CompilerParams(
            dimension_semantics=("parallel",),
        ),
    )(embs)


# speedup vs baseline: 2.3214x; 1.2724x over previous
import jax
import jax.numpy as jnp
from jax.experimental import pallas as pl
from jax.experimental.pallas import tpu as pltpu

_ROWS = 10000
_FEAT = 256
_BLOCK_ROWS = 5000


def _gelu_exact(x):
    return 0.5 * x * (1.0 + jax.lax.erf(x * 0.7071067811865476))


def _double_gelu_kernel(x_ref, o_ref):
    o_ref[...] = _gelu_exact(_gelu_exact(x_ref[...]))


def kernel(embs, edge_index, batch_size):
    del edge_index, batch_size
    n, d = embs.shape
    block_rows = _BLOCK_ROWS if (n == _ROWS and d == _FEAT) else n
    grid = (n // block_rows,) if n % block_rows == 0 else (1,)
    if n % block_rows != 0:
        block_rows = n
    return pl.pallas_call(
        _double_gelu_kernel,
        grid=grid,
        in_specs=[pl.BlockSpec((block_rows, d), lambda i: (i, 0))],
        out_specs=pl.BlockSpec((block_rows, d), lambda i: (i, 0)),
        out_shape=jax.ShapeDtypeStruct((n, d), embs.dtype),
        compiler_params=pltpu.CompilerParams(
            dimension_semantics=("parallel",),
        ),
    )(embs)
